# baseline (speedup 1.0000x reference)
import functools

import jax
import jax.numpy as jnp
from jax import lax
from jax.experimental import pallas as pl
from jax.experimental.pallas import tpu as pltpu
from jax.experimental.pallas import tpu_sc as plsc

_NC = 2
_NS = 16
_NW = _NC * _NS
_CH = 128
_LANES = 16


def _deg_body(n_pad, per_w, tail, dst_hbm, out_hbm,
              idx_v, ones_v, zeros_v, deg_sh):
    c = lax.axis_index("c")
    s = lax.axis_index("s")
    wid = c * _NS + s
    tile_slice = n_pad // _NS

    ones16 = jnp.ones((_LANES,), jnp.float32)
    zeros16 = jnp.zeros((_LANES,), jnp.float32)
    for i in range(_CH // _LANES):
        ones_v[pl.ds(i * _LANES, _LANES)] = ones16
    for i in range(tile_slice // _LANES):
        zeros_v[pl.ds(i * _LANES, _LANES)] = zeros16

    pltpu.sync_copy(zeros_v, deg_sh.at[pl.ds(s * tile_slice, tile_slice)])

    @pl.when(wid < _NW - 1)
    def _stage_full():
        pltpu.sync_copy(dst_hbm.at[1, pl.ds(wid * per_w * _CH, per_w * _CH)],
                        idx_v)

    @pl.when(wid == _NW - 1)
    def _stage_tail():
        pltpu.sync_copy(
            dst_hbm.at[1, pl.ds((_NW - 1) * per_w * _CH, tail * _CH)],
            idx_v.at[pl.ds(0, tail * _CH)])

    plsc.subcore_barrier()

    def chunk(j, carry):
        pltpu.sync_copy(ones_v, deg_sh.at[idx_v.at[pl.ds(j * _CH, _CH)]],
                        add=True)
        return carry

    n_my = jnp.where(wid < _NW - 1, per_w, tail)
    lax.fori_loop(0, n_my, chunk, 0)
    plsc.subcore_barrier()

    pltpu.sync_copy(
        deg_sh.at[pl.ds(s * tile_slice, tile_slice)],
        out_hbm.at[c, pl.ds(s * tile_slice, tile_slice)],
    )


def _degree_counts(edge_index, n_pad):
    n_edges = edge_index.shape[1]
    n_chunks = n_edges // _CH
    per_w = (-(-n_chunks // _NW) + 7) // 8 * 8
    tail = n_chunks - (_NW - 1) * per_w

    mesh = plsc.VectorSubcoreMesh(core_axis_name="c", subcore_axis_name="s")
    call = pl.kernel(
        functools.partial(_deg_body, n_pad, per_w, tail),
        out_type=jax.ShapeDtypeStruct((_NC, n_pad), jnp.float32),
        mesh=mesh,
        scratch_types=[
            pltpu.VMEM((per_w * _CH,), jnp.int32),
            pltpu.VMEM((_CH,), jnp.float32),
            pltpu.VMEM((n_pad // _NS,), jnp.float32),
            pltpu.VMEM_SHARED((n_pad,), jnp.float32),
        ],
    )
    return call(edge_index)


def _matmul_body(x_ref, w_ref, b_ref, y_ref):
    y = jnp.dot(x_ref[...], w_ref[...], preferred_element_type=jnp.float32)
    y_ref[...] = jnp.maximum(y + b_ref[...], 0.0)


def _mask_body(n_nodes, y_ref, deg_ref, o_ref):
    deg = jnp.sum(deg_ref[...], axis=0)
    deg = deg.reshape(-1, 1)[:n_nodes]
    o_ref[...] = jnp.where(deg > 0.0, y_ref[...], 0.0)


def kernel(x, p_t, edge_index, edge_weights, W_value, b_value):
    n_nodes, _ = x.shape
    out_ch = W_value.shape[1]
    n_pad = -(-n_nodes // (_NS * _LANES)) * (_NS * _LANES)

    deg2 = _degree_counts(edge_index, n_pad)

    y = pl.pallas_call(
        _matmul_body,
        out_shape=jax.ShapeDtypeStruct((n_nodes, out_ch), jnp.float32),
    )(x, W_value, b_value.reshape(1, -1))

    out = pl.pallas_call(
        functools.partial(_mask_body, n_nodes),
        out_shape=jax.ShapeDtypeStruct((n_nodes, out_ch), jnp.float32),
        input_output_aliases={0: 0},
    )(y, deg2)
    return out


# speedup vs baseline: 1.0482x; 1.0482x over previous
import functools

import jax
import jax.numpy as jnp
from jax import lax
from jax.experimental import pallas as pl
from jax.experimental.pallas import tpu as pltpu
from jax.experimental.pallas import tpu_sc as plsc

_NC = 2
_NS = 16
_NW = _NC * _NS
_CH = 128
_LANES = 16


def _deg_body(n_pad, per_w, tail, dst_hbm, out_hbm,
              idx_v, ones_v, zeros_v, deg_sh):
    c = lax.axis_index("c")
    s = lax.axis_index("s")
    wid = c * _NS + s
    tile_slice = n_pad // _NS

    ones16 = jnp.ones((_LANES,), jnp.float32)
    zeros16 = jnp.zeros((_LANES,), jnp.float32)
    for i in range(_CH // _LANES):
        ones_v[pl.ds(i * _LANES, _LANES)] = ones16
    for i in range(tile_slice // _LANES):
        zeros_v[pl.ds(i * _LANES, _LANES)] = zeros16

    pltpu.sync_copy(zeros_v, deg_sh.at[pl.ds(s * tile_slice, tile_slice)])

    @pl.when(wid < _NW - 1)
    def _stage_full():
        pltpu.sync_copy(dst_hbm.at[1, pl.ds(wid * per_w * _CH, per_w * _CH)],
                        idx_v)

    @pl.when(wid == _NW - 1)
    def _stage_tail():
        pltpu.sync_copy(
            dst_hbm.at[1, pl.ds((_NW - 1) * per_w * _CH, tail * _CH)],
            idx_v.at[pl.ds(0, tail * _CH)])

    plsc.subcore_barrier()

    def chunk(j, carry):
        pltpu.sync_copy(ones_v, deg_sh.at[idx_v.at[pl.ds(j * _CH, _CH)]],
                        add=True)
        return carry

    n_my = jnp.where(wid < _NW - 1, per_w, tail)
    lax.fori_loop(0, n_my, chunk, 0)
    plsc.subcore_barrier()

    pltpu.sync_copy(
        deg_sh.at[pl.ds(s * tile_slice, tile_slice)],
        out_hbm.at[c, pl.ds(s * tile_slice, tile_slice)],
    )


def _degree_counts(edge_index, n_pad):
    n_edges = edge_index.shape[1]
    n_chunks = n_edges // _CH
    per_w = (-(-n_chunks // _NW) + 7) // 8 * 8
    tail = n_chunks - (_NW - 1) * per_w

    mesh = plsc.VectorSubcoreMesh(core_axis_name="c", subcore_axis_name="s")
    call = pl.kernel(
        functools.partial(_deg_body, n_pad, per_w, tail),
        out_type=jax.ShapeDtypeStruct((_NC, n_pad), jnp.float32),
        mesh=mesh,
        scratch_types=[
            pltpu.VMEM((per_w * _CH,), jnp.int32),
            pltpu.VMEM((_CH,), jnp.float32),
            pltpu.VMEM((n_pad // _NS,), jnp.float32),
            pltpu.VMEM_SHARED((n_pad,), jnp.float32),
        ],
    )
    return call(edge_index)


def _dense_body(n_nodes, x_ref, w_ref, b_ref, deg_ref, o_ref):
    y = jnp.dot(x_ref[...], w_ref[...], preferred_element_type=jnp.float32)
    y = jnp.maximum(y + b_ref[...], 0.0)
    deg = jnp.sum(deg_ref[...], axis=0)
    deg = deg.reshape(-1, 1)[:n_nodes]
    o_ref[...] = jnp.where(deg > 0.0, y, 0.0)


def kernel(x, p_t, edge_index, edge_weights, W_value, b_value):
    n_nodes, _ = x.shape
    out_ch = W_value.shape[1]
    n_pad = -(-n_nodes // (_NS * _LANES)) * (_NS * _LANES)

    deg2 = _degree_counts(edge_index, n_pad)

    out = pl.pallas_call(
        functools.partial(_dense_body, n_nodes),
        out_shape=jax.ShapeDtypeStruct((n_nodes, out_ch), jnp.float32),
    )(x, W_value, b_value.reshape(1, -1), deg2)
    return out
